# baseline (speedup 1.0000x reference)
import functools

import jax
import jax.numpy as jnp
from jax import lax
from jax.experimental import pallas as pl
from jax.experimental.pallas import tpu as pltpu
from jax.experimental.pallas import tpu_sc as plsc

_CH = 4


def _tc_small_kernel(v_ref, m_ref, vo_ref, mo_ref):
    vo_ref[...] = v_ref[...]
    mo_ref[...] = m_ref[...] != 0.0


def _sc_copy_x(x):
    B, H, L = x.shape
    info = plsc.get_sparse_core_info()
    nw = info.num_cores * info.num_subcores
    hpw = (B * H) // nw
    rows = hpw // _CH
    wpb = H // hpw
    mesh = plsc.VectorSubcoreMesh(core_axis_name="c", subcore_axis_name="s")

    @functools.partial(
        pl.kernel, mesh=mesh,
        out_type=jax.ShapeDtypeStruct(x.shape, x.dtype),
        scratch_types=[
            pltpu.VMEM((2, rows, L), x.dtype),
            pltpu.SemaphoreType.DMA,
            pltpu.SemaphoreType.DMA,
            pltpu.SemaphoreType.DMA,
            pltpu.SemaphoreType.DMA,
        ],
    )
    def k(x_hbm, o_hbm, buf, si0, si1, so0, so1):
        wid = lax.axis_index("s") * info.num_cores + lax.axis_index("c")
        b = wid // wpb
        h0 = (wid % wpb) * hpw
        sin = (si0, si1)
        sout = (so0, so1)

        def in_copy(ci):
            return pltpu.make_async_copy(
                x_hbm.at[b, pl.ds(h0 + ci * rows, rows)],
                buf.at[ci % 2], sin[ci % 2])

        def out_copy(ci):
            return pltpu.make_async_copy(
                buf.at[ci % 2],
                o_hbm.at[b, pl.ds(h0 + ci * rows, rows)], sout[ci % 2])

        in_copy(0).start()
        in_copy(1).start()
        for ci in range(_CH):
            in_copy(ci).wait()
            out_copy(ci).start()
            if ci + 2 < _CH:
                out_copy(ci).wait()
                in_copy(ci + 2).start()
        out_copy(_CH - 2).wait()
        out_copy(_CH - 1).wait()

    return k(x)


def kernel(x, v, mask):
    x_out = _sc_copy_x(x)
    v_out, m_out = pl.pallas_call(
        _tc_small_kernel,
        in_specs=[
            pl.BlockSpec(v.shape, lambda: (0, 0, 0)),
            pl.BlockSpec(mask.shape, lambda: (0, 0, 0)),
        ],
        out_specs=[
            pl.BlockSpec(v.shape, lambda: (0, 0, 0)),
            pl.BlockSpec(mask.shape, lambda: (0, 0, 0)),
        ],
        out_shape=[
            jax.ShapeDtypeStruct(v.shape, v.dtype),
            jax.ShapeDtypeStruct(mask.shape, jnp.bool_),
        ],
    )(v, mask)
    return (x_out, v_out, m_out)


# speedup vs baseline: 1.1473x; 1.1473x over previous
import functools

import jax
import jax.numpy as jnp
from jax import lax
from jax.experimental import pallas as pl
from jax.experimental.pallas import tpu as pltpu
from jax.experimental.pallas import tpu_sc as plsc

_GRID = 2


def _tc_copy_kernel(x_ref, m_ref, xo_ref, mo_ref):
    xo_ref[...] = x_ref[...]

    @pl.when(pl.program_id(0) == 0)
    def _():
        mo_ref[...] = m_ref[...] != 0.0


def _sc_copy_v(v):
    B, H, L = v.shape
    info = plsc.get_sparse_core_info()
    nw = info.num_cores * info.num_subcores
    rpw = (B * H) // nw
    wpb = H // rpw
    mesh = plsc.VectorSubcoreMesh(core_axis_name="c", subcore_axis_name="s")

    @functools.partial(
        pl.kernel, mesh=mesh,
        out_type=jax.ShapeDtypeStruct(v.shape, v.dtype),
        scratch_types=[pltpu.VMEM((rpw, L), v.dtype)],
    )
    def k(v_hbm, o_hbm, buf):
        wid = lax.axis_index("s") * info.num_cores + lax.axis_index("c")
        b = wid // wpb
        h0 = (wid % wpb) * rpw
        pltpu.sync_copy(v_hbm.at[b, pl.ds(h0, rpw)], buf)
        pltpu.sync_copy(buf, o_hbm.at[b, pl.ds(h0, rpw)])

    return k(v)


def kernel(x, v, mask):
    v_out = _sc_copy_v(v)
    B, H, L = x.shape
    xspec = pl.BlockSpec((B // _GRID, H, L), lambda i: (i, 0, 0))
    x_out, m_out = pl.pallas_call(
        _tc_copy_kernel,
        grid=(_GRID,),
        in_specs=[
            xspec,
            pl.BlockSpec(mask.shape, lambda i: (0, 0, 0)),
        ],
        out_specs=[
            xspec,
            pl.BlockSpec(mask.shape, lambda i: (0, 0, 0)),
        ],
        out_shape=[
            jax.ShapeDtypeStruct(x.shape, x.dtype),
            jax.ShapeDtypeStruct(mask.shape, jnp.bool_),
        ],
    )(x, mask)
    return (x_out, v_out, m_out)


# speedup vs baseline: 2.2006x; 1.9181x over previous
import jax
import jax.numpy as jnp
from jax.experimental import pallas as pl
from jax.experimental.pallas import tpu as pltpu

_NCHUNK = 4


def _trim_passthrough_kernel(x_hbm, v_ref, m_ref, xo_hbm, vo_ref, mo_ref,
                             buf, sem_in, sem_out):
    rows = x_hbm.shape[0] // _NCHUNK

    def in_copy(i):
        return pltpu.make_async_copy(x_hbm.at[pl.ds(i * rows, rows)],
                                     buf.at[i], sem_in.at[i])

    def out_copy(i):
        return pltpu.make_async_copy(buf.at[i],
                                     xo_hbm.at[pl.ds(i * rows, rows)],
                                     sem_out.at[i])

    for i in range(_NCHUNK):
        in_copy(i).start()

    vo_ref[...] = v_ref[...]
    mo_ref[...] = m_ref[...] != 0.0

    for i in range(_NCHUNK):
        in_copy(i).wait()
        out_copy(i).start()
    for i in range(_NCHUNK):
        out_copy(i).wait()


def kernel(x, v, mask):
    B, H, L = x.shape
    out = pl.pallas_call(
        _trim_passthrough_kernel,
        in_specs=[
            pl.BlockSpec(memory_space=pl.ANY),
            pl.BlockSpec(v.shape, lambda: (0, 0, 0)),
            pl.BlockSpec(mask.shape, lambda: (0, 0, 0)),
        ],
        out_specs=[
            pl.BlockSpec(memory_space=pl.ANY),
            pl.BlockSpec(v.shape, lambda: (0, 0, 0)),
            pl.BlockSpec(mask.shape, lambda: (0, 0, 0)),
        ],
        out_shape=[
            jax.ShapeDtypeStruct(x.shape, x.dtype),
            jax.ShapeDtypeStruct(v.shape, v.dtype),
            jax.ShapeDtypeStruct(mask.shape, jnp.bool_),
        ],
        scratch_shapes=[
            pltpu.VMEM((_NCHUNK, B // _NCHUNK, H, L), x.dtype),
            pltpu.SemaphoreType.DMA((_NCHUNK,)),
            pltpu.SemaphoreType.DMA((_NCHUNK,)),
        ],
    )(x, v, mask)
    return (out[0], out[1], out[2])


# speedup vs baseline: 2.4503x; 1.1135x over previous
import jax
import jax.numpy as jnp
from jax.experimental import pallas as pl
from jax.experimental.pallas import tpu as pltpu

_GRID = 2


def _trim_passthrough_kernel(x_ref, v_ref, m_ref, xo_ref, vo_ref, mo_ref):
    xo_ref[...] = x_ref[...]

    @pl.when(pl.program_id(0) == 0)
    def _():
        vo_ref[...] = v_ref[...]
        mo_ref[...] = m_ref[...] != 0.0


def kernel(x, v, mask):
    B, H, L = x.shape
    b = B // _GRID
    xspec = pl.BlockSpec((b, H, L), lambda i: (i, 0, 0))
    out = pl.pallas_call(
        _trim_passthrough_kernel,
        grid=(_GRID,),
        in_specs=[
            xspec,
            pl.BlockSpec(v.shape, lambda i: (0, 0, 0)),
            pl.BlockSpec(mask.shape, lambda i: (0, 0, 0)),
        ],
        out_specs=[
            xspec,
            pl.BlockSpec(v.shape, lambda i: (0, 0, 0)),
            pl.BlockSpec(mask.shape, lambda i: (0, 0, 0)),
        ],
        out_shape=[
            jax.ShapeDtypeStruct(x.shape, x.dtype),
            jax.ShapeDtypeStruct(v.shape, v.dtype),
            jax.ShapeDtypeStruct(mask.shape, jnp.bool_),
        ],
    )(x, v, mask)
    return (out[0], out[1], out[2])
